# Initial kernel scaffold; baseline (speedup 1.0000x reference)
import jax
import jax.numpy as jnp
from jax.experimental import pallas as pl


def kernel(g_emb, g_candidates_emb, batch_idx, W0, b0, W1, b1, Wf, bf):
    raise NotImplementedError("write your pallas kernel here")


# speedup vs baseline: 4.8961x; 4.8961x over previous
import functools

import jax
import jax.numpy as jnp
from jax import lax
from jax.experimental import pallas as pl
from jax.experimental.pallas import tpu as pltpu
from jax.experimental.pallas import tpu_sc as plsc

B = 4096
N = 204800
EMB = 64
HID = 128

NC = 2
NS = 16
NW = NC * NS
ROWS_W = N // NW
CH = 128
NCH = ROWS_W // CH
NCHUNKS = N // CH
CH_SC = NCHUNKS // NS

_mesh = plsc.VectorSubcoreMesh(core_axis_name="c", subcore_axis_name="s")


@functools.partial(
    pl.kernel,
    mesh=_mesh,
    out_type=jax.ShapeDtypeStruct((N, EMB), jnp.float32),
    scratch_types=[
        pltpu.VMEM((NCH, CH), jnp.int32),
        pltpu.VMEM((CH, EMB), jnp.float32),
        pltpu.VMEM((CH, EMB), jnp.float32),
        pltpu.SemaphoreType.DMA,
        pltpu.SemaphoreType.DMA,
    ],
    compiler_params=pltpu.CompilerParams(use_tc_tiling_on_sc=False),
)
def _gather_rep(emb_hbm, idx_hbm, out_hbm, idx_v, buf0, buf1, sem0, sem1):
    c = lax.axis_index("c")
    s = lax.axis_index("s")
    wid = s * NC + c
    base = wid * ROWS_W
    pltpu.sync_copy(idx_hbm.at[wid], idx_v)

    def fire(j, buf, sem):
        pltpu.async_copy(emb_hbm.at[idx_v.at[j]], buf, sem)

    def drain(j, buf, sem):
        pltpu.make_async_copy(emb_hbm.at[idx_v.at[j]], buf, sem).wait()
        pltpu.sync_copy(buf, out_hbm.at[pl.ds(base + j * CH, CH)])

    fire(0, buf0, sem0)

    def body(i, carry):
        j0 = 2 * i
        fire(j0 + 1, buf1, sem1)
        drain(j0, buf0, sem0)

        @pl.when(j0 + 2 < NCH)
        def _():
            fire(j0 + 2, buf0, sem0)

        drain(j0 + 1, buf1, sem1)
        return carry

    lax.fori_loop(0, NCH // 2, body, 0)


TILE = 2048


def _mlp_body(xr_ref, gc_ref, w0a_ref, w0b_ref, b0_ref, w1_ref, b1_ref,
              wf_ref, bf_ref, xs_ref, ex_ref):
    xr = xr_ref[...]
    gc = gc_ref[...]
    xs_ref[:, :EMB] = xr
    xs_ref[:, EMB:] = gc
    h = jnp.dot(xr, w0a_ref[...], preferred_element_type=jnp.float32)
    h += jnp.dot(gc, w0b_ref[...], preferred_element_type=jnp.float32)
    h = jnp.maximum(h + b0_ref[...], 0.0)
    h = jnp.dot(h, w1_ref[...], preferred_element_type=jnp.float32)
    h = jnp.maximum(h + b1_ref[...], 0.0)
    logits = jnp.sum(h * wf_ref[...], axis=1, keepdims=True) + bf_ref[...]
    ex_ref[...] = jnp.exp(logits)


_mlp = pl.pallas_call(
    _mlp_body,
    grid=(N // TILE,),
    in_specs=[
        pl.BlockSpec((TILE, EMB), lambda i: (i, 0)),
        pl.BlockSpec((TILE, EMB), lambda i: (i, 0)),
        pl.BlockSpec((EMB, HID), lambda i: (0, 0)),
        pl.BlockSpec((EMB, HID), lambda i: (0, 0)),
        pl.BlockSpec((1, HID), lambda i: (0, 0)),
        pl.BlockSpec((HID, HID), lambda i: (0, 0)),
        pl.BlockSpec((1, HID), lambda i: (0, 0)),
        pl.BlockSpec((1, HID), lambda i: (0, 0)),
        pl.BlockSpec((1, 1), lambda i: (0, 0)),
    ],
    out_specs=[
        pl.BlockSpec((TILE, HID), lambda i: (i, 0)),
        pl.BlockSpec((TILE, 1), lambda i: (i, 0)),
    ],
    out_shape=[
        jax.ShapeDtypeStruct((N, HID), jnp.float32),
        jax.ShapeDtypeStruct((N, 1), jnp.float32),
    ],
)


@functools.partial(
    pl.kernel,
    mesh=_mesh,
    out_type=jax.ShapeDtypeStruct((NW, NCH, CH), jnp.float32),
    scratch_types=[
        pltpu.VMEM((CH_SC, CH), jnp.int32),
        pltpu.VMEM((CH_SC, CH), jnp.float32),
        pltpu.VMEM((NCH, CH), jnp.int32),
        pltpu.VMEM((NCH, CH), jnp.float32),
        pltpu.VMEM((NCH, CH), jnp.float32),
        pltpu.VMEM((B,), jnp.float32),
        pltpu.VMEM_SHARED((B,), jnp.float32),
    ],
    compiler_params=pltpu.CompilerParams(needs_layout_passes=False),
)
def _seg_softmax(ex_sc_hbm, idx_sc_hbm, ex_hbm, idx_hbm, out_hbm,
                 idx_a, ex_a, idx_b, ex_b, out_v, table_v, table_sh):
    c = lax.axis_index("c")
    s = lax.axis_index("s")
    wid = s * NC + c

    @pl.when(s == 0)
    def _():
        def zbody(i, carry):
            table_v[pl.ds(i * 16, 16)] = jnp.zeros((16,), jnp.float32)
            return carry
        lax.fori_loop(0, B // 16, zbody, 0)
        pltpu.sync_copy(table_v, table_sh)

    plsc.subcore_barrier()

    pltpu.sync_copy(idx_sc_hbm.at[s], idx_a)
    pltpu.sync_copy(ex_sc_hbm.at[s], ex_a)

    def sbody(j, carry):
        pltpu.sync_copy(ex_a.at[j], table_sh.at[idx_a.at[j]], add=True)
        return carry

    lax.fori_loop(0, CH_SC, sbody, 0)
    plsc.subcore_barrier()

    pltpu.sync_copy(table_sh, table_v)
    pltpu.sync_copy(idx_hbm.at[wid], idx_b)
    pltpu.sync_copy(ex_hbm.at[wid], ex_b)

    def dbody(j, carry):
        def inner(k, carry2):
            idx16 = idx_b[j, pl.ds(k * 16, 16)]
            ex16 = ex_b[j, pl.ds(k * 16, 16)]
            den16 = plsc.load_gather(table_v, [idx16])
            out_v[j, pl.ds(k * 16, 16)] = ex16 / den16
            return carry2
        return lax.fori_loop(0, CH // 16, inner, carry)

    lax.fori_loop(0, NCH, dbody, 0)
    pltpu.sync_copy(out_v, out_hbm.at[wid])


def kernel(g_emb, g_candidates_emb, batch_idx, W0, b0, W1, b1, Wf, bf):
    idx3 = batch_idx.reshape(NW, NCH, CH)
    idx_sc = batch_idx.reshape(NS, CH_SC, CH)
    x_rep = _gather_rep(g_emb, idx3)
    x_states, ex = _mlp(
        x_rep, g_candidates_emb,
        W0[:EMB], W0[EMB:],
        b0.reshape(1, HID), W1, b1.reshape(1, HID),
        Wf.reshape(1, HID), bf.reshape(1, 1),
    )
    exf = ex.reshape(N)
    probs = _seg_softmax(exf.reshape(NS, CH_SC, CH), idx_sc,
                         exf.reshape(NW, NCH, CH), idx3)
    return (g_emb, x_states, probs.reshape(N))


# speedup vs baseline: 7.7269x; 1.5782x over previous
import functools

import jax
import jax.numpy as jnp
from jax import lax
from jax.experimental import pallas as pl
from jax.experimental.pallas import tpu as pltpu
from jax.experimental.pallas import tpu_sc as plsc

B = 4096
N = 204800
EMB = 64
HID = 128

NC = 2
NS = 16
NW = NC * NS
ROWS_W = N // NW
CH = 128
NCH = ROWS_W // CH
NCHUNKS = N // CH
CH_SC = NCHUNKS // NS

_mesh = plsc.VectorSubcoreMesh(core_axis_name="c", subcore_axis_name="s")


@functools.partial(
    pl.kernel,
    mesh=_mesh,
    out_type=jax.ShapeDtypeStruct((N, HID), jnp.float32),
    scratch_types=[
        pltpu.VMEM((NCH, CH), jnp.int32),
        pltpu.VMEM((4, CH, EMB), jnp.float32),
        pltpu.SemaphoreType.DMA((4,)),
    ],
    compiler_params=pltpu.CompilerParams(use_tc_tiling_on_sc=False),
)
def _gather_rep(emb_hbm, idx_hbm, out_hbm, idx_v, buf_v, sems):
    c = lax.axis_index("c")
    s = lax.axis_index("s")
    wid = s * NC + c
    base = wid * ROWS_W
    pltpu.sync_copy(idx_hbm.at[wid], idx_v)

    def fire(j):
        sl = lax.rem(j, 4)
        pltpu.async_copy(emb_hbm.at[idx_v.at[j]], buf_v.at[sl], sems.at[sl])

    for j in range(3):
        fire(j)

    def body(j, carry):
        sl = lax.rem(j, 4)
        pltpu.make_async_copy(
            emb_hbm.at[idx_v.at[j]], buf_v.at[sl], sems.at[sl]).wait()
        pltpu.sync_copy(buf_v.at[sl],
                        out_hbm.at[pl.ds(base + j * CH, CH), pl.ds(0, EMB)])

        @pl.when(j + 3 < NCH)
        def _():
            fire(j + 3)

        return carry

    lax.fori_loop(0, NCH, body, 0)


TILE = 2048


def _mlp_body(xs_ref, gcT_ref, w0a_ref, w0b_ref, b0_ref, w1_ref, b1_ref,
              wf_ref, bf_ref, eye_ref, xs_out, ex_ref):
    f32 = jnp.float32
    xr = xs_ref[:, :EMB]
    gcT = gcT_ref[...]
    gc = jax.lax.dot_general(gcT, eye_ref[...], (((0,), (0,)), ((), ())),
                             preferred_element_type=f32)
    xs_out[:, :EMB] = xr
    xs_out[:, EMB:] = gc
    h = jnp.dot(xr, w0a_ref[...], preferred_element_type=f32)
    h += jax.lax.dot_general(gcT, w0b_ref[...], (((0,), (0,)), ((), ())),
                             preferred_element_type=f32)
    h = jnp.maximum(h + b0_ref[...], 0.0)
    h = jnp.dot(h, w1_ref[...], preferred_element_type=f32)
    h = jnp.maximum(h + b1_ref[...], 0.0)
    logits = jnp.sum(h * wf_ref[...], axis=1) + bf_ref[0, 0]
    ex_ref[...] = jnp.exp(logits).reshape(TILE // CH, CH)


_mlp = pl.pallas_call(
    _mlp_body,
    grid=(N // TILE,),
    in_specs=[
        pl.BlockSpec((TILE, HID), lambda i: (i, 0)),
        pl.BlockSpec((EMB, TILE), lambda i: (0, i)),
        pl.BlockSpec((EMB, HID), lambda i: (0, 0)),
        pl.BlockSpec((EMB, HID), lambda i: (0, 0)),
        pl.BlockSpec((1, HID), lambda i: (0, 0)),
        pl.BlockSpec((HID, HID), lambda i: (0, 0)),
        pl.BlockSpec((1, HID), lambda i: (0, 0)),
        pl.BlockSpec((1, HID), lambda i: (0, 0)),
        pl.BlockSpec((1, 1), lambda i: (0, 0)),
        pl.BlockSpec((EMB, EMB), lambda i: (0, 0)),
    ],
    out_specs=[
        pl.BlockSpec((TILE, HID), lambda i: (i, 0)),
        pl.BlockSpec((TILE // CH, CH), lambda i: (i, 0)),
    ],
    out_shape=[
        jax.ShapeDtypeStruct((N, HID), jnp.float32),
        jax.ShapeDtypeStruct((NCHUNKS, CH), jnp.float32),
    ],
    input_output_aliases={0: 0},
)


@functools.partial(
    pl.kernel,
    mesh=_mesh,
    out_type=jax.ShapeDtypeStruct((NW, NCH, CH), jnp.float32),
    scratch_types=[
        pltpu.VMEM((CH_SC, CH), jnp.int32),
        pltpu.VMEM((CH_SC, CH), jnp.float32),
        pltpu.VMEM((NCH, CH), jnp.int32),
        pltpu.VMEM((NCH, CH), jnp.float32),
        pltpu.VMEM((NCH, CH), jnp.float32),
        pltpu.VMEM((B,), jnp.float32),
        pltpu.VMEM_SHARED((B,), jnp.float32),
    ],
    compiler_params=pltpu.CompilerParams(needs_layout_passes=False),
)
def _seg_softmax(ex_sc_hbm, idx_sc_hbm, ex_hbm, idx_hbm, out_hbm,
                 idx_a, ex_a, idx_b, ex_b, out_v, table_v, table_sh):
    c = lax.axis_index("c")
    s = lax.axis_index("s")
    wid = s * NC + c

    @pl.when(s == 0)
    def _():
        def zbody(i, carry):
            table_v[pl.ds(i * 16, 16)] = jnp.zeros((16,), jnp.float32)
            return carry
        lax.fori_loop(0, B // 16, zbody, 0)
        pltpu.sync_copy(table_v, table_sh)

    plsc.subcore_barrier()

    pltpu.sync_copy(idx_sc_hbm.at[s], idx_a)
    pltpu.sync_copy(ex_sc_hbm.at[s], ex_a)

    def sbody(j, carry):
        pltpu.sync_copy(ex_a.at[j], table_sh.at[idx_a.at[j]], add=True)
        return carry

    lax.fori_loop(0, CH_SC, sbody, 0)
    plsc.subcore_barrier()

    pltpu.sync_copy(table_sh, table_v)
    pltpu.sync_copy(idx_hbm.at[wid], idx_b)
    pltpu.sync_copy(ex_hbm.at[wid], ex_b)

    def dbody(j, carry):
        def inner(k, carry2):
            idx16 = idx_b[j, pl.ds(k * 16, 16)]
            ex16 = ex_b[j, pl.ds(k * 16, 16)]
            den16 = plsc.load_gather(table_v, [idx16])
            out_v[j, pl.ds(k * 16, 16)] = ex16 / den16
            return carry2
        return lax.fori_loop(0, CH // 16, inner, carry)

    lax.fori_loop(0, NCH, dbody, 0)
    pltpu.sync_copy(out_v, out_hbm.at[wid])


def kernel(g_emb, g_candidates_emb, batch_idx, W0, b0, W1, b1, Wf, bf):
    idx3 = batch_idx.reshape(NW, NCH, CH)
    idx_sc = batch_idx.reshape(NS, CH_SC, CH)
    xs0 = _gather_rep(g_emb, idx3)
    x_states, ex = _mlp(
        xs0, g_candidates_emb.T,
        W0[:EMB], W0[EMB:],
        b0.reshape(1, HID), W1, b1.reshape(1, HID),
        Wf.reshape(1, HID), bf.reshape(1, 1),
        jnp.eye(EMB, dtype=jnp.float32),
    )
    probs = _seg_softmax(ex.reshape(NS, CH_SC, CH), idx_sc,
                         ex.reshape(NW, NCH, CH), idx3)
    return (g_emb, x_states, probs.reshape(N))
